# baseline (device time: 74486 ns/iter reference)
import functools

import jax
import jax.numpy as jnp
from jax import lax
from jax.experimental import pallas as pl
from jax.experimental.pallas import tpu as pltpu

N_DEV = 16
B, S, D = 2, 256, 1024
H, Dh, Dr = 16, 64, 32
BS = B * S
ROWS = 2 * BS
Q4 = ROWS // 4
HQ = Q4 // 2
SUB = HQ // 4

BF = jnp.bfloat16
F32 = jnp.float32
MESH = pl.DeviceIdType.MESH


def kernel(x, Wdkv, Wuk, Wuv, Wq, Wqr, Wkr, Wo):
    def body(x_ref, wdkv_ref, wuk_ref, wuv_ref, wq_ref, wqr_ref, wkr_ref,
             wo_ref, out_ref,
             acc_ref, kv_ref, o_ref, rope_ref,
             prs_send, prs_recv, zrs_send1, zrs_recv1, zrs_send2, zrs_recv2,
             zag_send1, zag_recv1, zag_send2, zag_recv2, pag_send, pag_recv,
             prs_ssem, prs_rsem, zrs_ssem, zrs_rsem,
             zag_ssem, zag_rsem, pag_ssem, pag_rsem):
        j = lax.axis_index("i")
        my_p = j // 4
        my_q = lax.rem(j, 4)
        base = my_p * 4
        plane_tgt = (base + lax.rem(my_q + 1, 4), base + lax.rem(my_q + 3, 4))
        b0 = lax.rem(my_p, 2)
        b1 = lax.rem(my_p // 2, 2)
        zh = (jnp.bitwise_xor(my_p, 1) * 4 + my_q,
              jnp.bitwise_xor(my_p, 2) * 4 + my_q)

        def pchunk(t, d):
            return pl.ds(t * Q4 + d * HQ, HQ)

        xb = x_ref[...].reshape(BS, D).astype(BF)
        c = jnp.dot(xb, wdkv_ref[...].astype(BF),
                    preferred_element_type=F32).astype(BF)
        acc_ref[0:BS, :] = jnp.dot(c, wuk_ref[...].astype(BF),
                                   preferred_element_type=F32)
        acc_ref[BS:ROWS, :] = jnp.dot(c, wuv_ref[...].astype(BF),
                                      preferred_element_type=F32)

        barrier_sem = pltpu.get_barrier_semaphore()
        for nbr in (*plane_tgt, *zh):
            pl.semaphore_signal(barrier_sem, inc=1, device_id=(nbr,),
                                device_id_type=MESH)
        pl.semaphore_wait(barrier_sem, 4)

        qm = qrm = krm = None
        scale = (Dh + Dr) ** -0.5
        nt = (((1,), (1,)), ((), ()))
        own_t = (lax.rem(my_q + 1, 4), lax.rem(my_q + 3, 4))
        Rb = (own_t[0] * Q4, own_t[1] * Q4 + HQ)

        for d in range(2):
            prs_send[d, 0] = acc_ref[pchunk(my_q, d), :].astype(BF)
        for s in range(3):
            rdmas = []
            for d in range(2):
                r = pltpu.make_async_remote_copy(
                    src_ref=prs_send.at[d, s], dst_ref=prs_recv.at[d, s],
                    send_sem=prs_ssem.at[d, s], recv_sem=prs_rsem.at[d, s],
                    device_id=(plane_tgt[d],), device_id_type=MESH)
                r.start()
                rdmas.append(r)
            if s == 0:
                qm = jnp.dot(xb, wq_ref[...].astype(BF),
                             preferred_element_type=F32).astype(BF)
            elif s == 1:
                qrm = jnp.dot(xb, wqr_ref[...].astype(BF),
                              preferred_element_type=F32).astype(BF)
                krm = jnp.dot(xb, wkr_ref[...].astype(BF),
                              preferred_element_type=F32).astype(BF)
            else:
                for h in range(H):
                    rope_ref[0:S, h * S:(h + 1) * S] = scale * lax.dot_general(
                        qrm[0:S, h * Dr:(h + 1) * Dr], krm[0:S, :], nt,
                        preferred_element_type=F32)
            for d in range(2):
                rdmas[d].wait()
            for d in range(2):
                t_r = (lax.rem(my_q - s - 1 + 8, 4) if d == 0
                       else lax.rem(my_q + s + 1, 4))
                tmp = acc_ref[pchunk(t_r, d), :] + prs_recv[d, s].astype(F32)
                acc_ref[pchunk(t_r, d), :] = tmp
                if s < 2:
                    prs_send[d, s + 1] = tmp.astype(BF)
                else:
                    zrs_send1[d] = acc_ref[
                        pl.ds(Rb[d] + (1 - b0) * 64, 64), :].astype(BF)

        rdmas = []
        for d in range(2):
            r = pltpu.make_async_remote_copy(
                src_ref=zrs_send1.at[d], dst_ref=zrs_recv1.at[d],
                send_sem=zrs_ssem.at[d, 0], recv_sem=zrs_rsem.at[d, 0],
                device_id=(zh[0],), device_id_type=MESH)
            r.start()
            rdmas.append(r)
        for h in range(H):
            rope_ref[S:BS, h * S:(h + 1) * S] = scale * lax.dot_general(
                qrm[S:BS, h * Dr:(h + 1) * Dr], krm[S:BS, :], nt,
                preferred_element_type=F32)
        for d in range(2):
            rdmas[d].wait()
        for d in range(2):
            keep1 = pl.ds(Rb[d] + b0 * 64, 64)
            tmp = acc_ref[keep1, :] + zrs_recv1[d].astype(F32)
            acc_ref[keep1, :] = tmp
            zrs_send2[d] = acc_ref[
                pl.ds(Rb[d] + b0 * 64 + (1 - b1) * 32, 32), :].astype(BF)
        rdmas = []
        for d in range(2):
            r = pltpu.make_async_remote_copy(
                src_ref=zrs_send2.at[d], dst_ref=zrs_recv2.at[d],
                send_sem=zrs_ssem.at[d, 1], recv_sem=zrs_rsem.at[d, 1],
                device_id=(zh[1],), device_id_type=MESH)
            r.start()
            rdmas.append(r)
        for d in range(2):
            rdmas[d].wait()
        for d in range(2):
            off = pl.ds(Rb[d] + b0 * 64 + b1 * 32, 32)
            tmp = (acc_ref[off, :] + zrs_recv2[d].astype(F32)).astype(BF)
            kv_ref[off, :] = tmp
            zag_send1[d] = tmp

        rdmas = []
        for d in range(2):
            r = pltpu.make_async_remote_copy(
                src_ref=zag_send1.at[d], dst_ref=zag_recv1.at[d],
                send_sem=zag_ssem.at[d, 0], recv_sem=zag_rsem.at[d, 0],
                device_id=(zh[1],), device_id_type=MESH)
            r.start()
            rdmas.append(r)
        for d in range(2):
            rdmas[d].wait()
        for d in range(2):
            kv_ref[pl.ds(Rb[d] + b0 * 64 + (1 - b1) * 32, 32), :] = zag_recv1[d]
            zag_send2[d] = kv_ref[pl.ds(Rb[d] + b0 * 64, 64), :]
        rdmas = []
        for d in range(2):
            r = pltpu.make_async_remote_copy(
                src_ref=zag_send2.at[d], dst_ref=zag_recv2.at[d],
                send_sem=zag_ssem.at[d, 1], recv_sem=zag_rsem.at[d, 1],
                device_id=(zh[0],), device_id_type=MESH)
            r.start()
            rdmas.append(r)
        for d in range(2):
            rdmas[d].wait()
        for d in range(2):
            kv_ref[pl.ds(Rb[d] + (1 - b0) * 64, 64), :] = zag_recv2[d]

        for d in range(2):
            pag_send[d, 0] = kv_ref[pchunk(own_t[d], d), :]
        for s in range(3):
            rdmas = []
            for d in range(2):
                src = pag_send.at[d, 0] if s == 0 else pag_recv.at[d, s - 1]
                r = pltpu.make_async_remote_copy(
                    src_ref=src, dst_ref=pag_recv.at[d, s],
                    send_sem=pag_ssem.at[d, s], recv_sem=pag_rsem.at[d, s],
                    device_id=(plane_tgt[d],), device_id_type=MESH)
                r.start()
                rdmas.append(r)
            if s > 0:
                for d in range(2):
                    t_got = (lax.rem(my_q - (s - 1) + 8, 4) if d == 0
                             else lax.rem(my_q + (s - 1), 4))
                    kv_ref[pchunk(t_got, d), :] = pag_recv[d, s - 1]
            for d in range(2):
                rdmas[d].wait()
        for d in range(2):
            t_got = lax.rem(my_q - 2 + 8, 4) if d == 0 else lax.rem(my_q + 2, 4)
            kv_ref[pchunk(t_got, d), :] = pag_recv[d, 2]

        K = kv_ref[0:BS, :]
        V = kv_ref[BS:ROWS, :]
        for b in range(B):
            for h in range(H):
                qh = qm[b * S:(b + 1) * S, h * Dh:(h + 1) * Dh]
                kh = K[b * S:(b + 1) * S, h * Dh:(h + 1) * Dh]
                vh = V[b * S:(b + 1) * S, h * Dh:(h + 1) * Dh]
                sc = (scale * lax.dot_general(qh, kh, nt,
                                              preferred_element_type=F32)
                      + rope_ref[b * S:(b + 1) * S, h * S:(h + 1) * S])
                pr = jnp.exp(sc)
                recip = 1.0 / jnp.sum(pr, axis=-1, keepdims=True)
                ov = jnp.dot(pr.astype(BF), vh, preferred_element_type=F32)
                o_ref[b * S:(b + 1) * S, h * Dh:(h + 1) * Dh] = (
                    ov * recip).astype(BF)
        out = jnp.dot(o_ref[...], wo_ref[...].astype(BF),
                      preferred_element_type=F32)
        out_ref[...] = out.reshape(B, S, D)

        @functools.partial(pl.run_scoped,
                           second_barrier=pltpu.SemaphoreType.REGULAR)
        def _(second_barrier):
            for nbr in (*plane_tgt, *zh):
                pl.semaphore_signal(second_barrier, inc=1, device_id=(nbr,),
                                    device_id_type=MESH)
            pl.semaphore_wait(second_barrier, 4)

    return pl.pallas_call(
        body,
        out_shape=jax.ShapeDtypeStruct((B, S, D), F32),
        in_specs=[pl.BlockSpec(memory_space=pltpu.VMEM)] * 8,
        out_specs=pl.BlockSpec(memory_space=pltpu.VMEM),
        scratch_shapes=[
            pltpu.VMEM((ROWS, D), F32),
            pltpu.VMEM((ROWS, D), BF),
            pltpu.VMEM((BS, D), BF),
            pltpu.VMEM((BS, H * S), F32),
            pltpu.VMEM((2, 3, HQ, D), BF),
            pltpu.VMEM((2, 3, HQ, D), BF),
            pltpu.VMEM((2, 64, D), BF),
            pltpu.VMEM((2, 64, D), BF),
            pltpu.VMEM((2, 32, D), BF),
            pltpu.VMEM((2, 32, D), BF),
            pltpu.VMEM((2, 32, D), BF),
            pltpu.VMEM((2, 32, D), BF),
            pltpu.VMEM((2, 64, D), BF),
            pltpu.VMEM((2, 64, D), BF),
            pltpu.VMEM((2, 1, HQ, D), BF),
            pltpu.VMEM((2, 3, HQ, D), BF),
            pltpu.SemaphoreType.DMA((2, 3)),
            pltpu.SemaphoreType.DMA((2, 3)),
            pltpu.SemaphoreType.DMA((2, 2)),
            pltpu.SemaphoreType.DMA((2, 2)),
            pltpu.SemaphoreType.DMA((2, 2)),
            pltpu.SemaphoreType.DMA((2, 2)),
            pltpu.SemaphoreType.DMA((2, 3)),
            pltpu.SemaphoreType.DMA((2, 3)),
        ],
        compiler_params=pltpu.CompilerParams(collective_id=0),
    )(x, Wdkv, Wuk, Wuv, Wq, Wqr, Wkr, Wo)


# device time: 72583 ns/iter; 1.0262x vs baseline; 1.0262x over previous
import functools

import jax
import jax.numpy as jnp
from jax import lax
from jax.experimental import pallas as pl
from jax.experimental.pallas import tpu as pltpu

N_DEV = 16
B, S, D = 2, 256, 1024
H, Dh, Dr = 16, 64, 32
BS = B * S
ROWS = 2 * BS
Q4 = ROWS // 4
HQ = Q4 // 2
SUB = HQ // 4

BF = jnp.bfloat16
F32 = jnp.float32
MESH = pl.DeviceIdType.MESH


def kernel(x, Wdkv, Wuk, Wuv, Wq, Wqr, Wkr, Wo):
    def body(x_ref, wdkv_ref, wuk_ref, wuv_ref, wq_ref, wqr_ref, wkr_ref,
             wo_ref, out_ref,
             acc_ref, kv_ref, o_ref, rope_ref,
             prs_send, prs_recv, zrs_send1, zrs_recv1, zrs_send2, zrs_recv2,
             zag_send1, zag_recv1, zag_send2, zag_recv2,
             pag_send, pag_recv1, pag_recv2,
             prs_ssem, prs_rsem, zrs_ssem, zrs_rsem,
             zag_ssem, zag_rsem, pag_ssem, pag_rsem):
        j = lax.axis_index("i")
        my_p = j // 4
        my_q = lax.rem(j, 4)
        base = my_p * 4
        plane_tgt = (base + lax.rem(my_q + 1, 4), base + lax.rem(my_q + 3, 4))
        b0 = lax.rem(my_p, 2)
        b1 = lax.rem(my_p // 2, 2)
        zh = (jnp.bitwise_xor(my_p, 1) * 4 + my_q,
              jnp.bitwise_xor(my_p, 2) * 4 + my_q)

        def pchunk(t, d):
            return pl.ds(t * Q4 + d * HQ, HQ)

        xb = x_ref[...].reshape(BS, D).astype(BF)
        c = jnp.dot(xb, wdkv_ref[...].astype(BF),
                    preferred_element_type=F32).astype(BF)
        acc_ref[0:BS, :] = jnp.dot(c, wuk_ref[...].astype(BF),
                                   preferred_element_type=F32)
        acc_ref[BS:ROWS, :] = jnp.dot(c, wuv_ref[...].astype(BF),
                                      preferred_element_type=F32)

        barrier_sem = pltpu.get_barrier_semaphore()
        for nbr in (*plane_tgt, *zh):
            pl.semaphore_signal(barrier_sem, inc=1, device_id=(nbr,),
                                device_id_type=MESH)
        pl.semaphore_wait(barrier_sem, 4)

        qm = qrm = krm = None
        scale = (Dh + Dr) ** -0.5
        nt = (((1,), (1,)), ((), ()))
        T = lax.rem(my_q + 1, 4)
        Rb = (T * Q4, T * Q4 + HQ)

        prs_send[0, 0] = acc_ref[pchunk(my_q, 0), :].astype(BF)
        prs_send[1, 0] = acc_ref[pchunk(lax.rem(my_q + 2, 4), 1), :].astype(BF)
        for s in range(3):
            rdmas = []
            for d in range(2):
                r = pltpu.make_async_remote_copy(
                    src_ref=prs_send.at[d, s], dst_ref=prs_recv.at[d, s],
                    send_sem=prs_ssem.at[d, s], recv_sem=prs_rsem.at[d, s],
                    device_id=(plane_tgt[d],), device_id_type=MESH)
                r.start()
                rdmas.append(r)
            if s == 0:
                qm = jnp.dot(xb, wq_ref[...].astype(BF),
                             preferred_element_type=F32).astype(BF)
            elif s == 1:
                qrm = jnp.dot(xb, wqr_ref[...].astype(BF),
                              preferred_element_type=F32).astype(BF)
                krm = jnp.dot(xb, wkr_ref[...].astype(BF),
                              preferred_element_type=F32).astype(BF)
            else:
                for h in range(H):
                    rope_ref[0:S, h * S:(h + 1) * S] = scale * lax.dot_general(
                        qrm[0:S, h * Dr:(h + 1) * Dr], krm[0:S, :], nt,
                        preferred_element_type=F32)
            for d in range(2):
                rdmas[d].wait()
            for d in range(2):
                t_r = (lax.rem(my_q - s - 1 + 8, 4) if d == 0
                       else lax.rem(my_q + s + 3, 4))
                tmp = acc_ref[pchunk(t_r, d), :] + prs_recv[d, s].astype(F32)
                acc_ref[pchunk(t_r, d), :] = tmp
                if s < 2:
                    prs_send[d, s + 1] = tmp.astype(BF)
                else:
                    zrs_send1[d] = acc_ref[
                        pl.ds(Rb[d] + (1 - b0) * 64, 64), :].astype(BF)

        rdmas = []
        for d in range(2):
            r = pltpu.make_async_remote_copy(
                src_ref=zrs_send1.at[d], dst_ref=zrs_recv1.at[d],
                send_sem=zrs_ssem.at[d, 0], recv_sem=zrs_rsem.at[d, 0],
                device_id=(zh[0],), device_id_type=MESH)
            r.start()
            rdmas.append(r)
        for h in range(H):
            rope_ref[S:BS, h * S:(h + 1) * S] = scale * lax.dot_general(
                qrm[S:BS, h * Dr:(h + 1) * Dr], krm[S:BS, :], nt,
                preferred_element_type=F32)
        for d in range(2):
            rdmas[d].wait()
        for d in range(2):
            keep1 = pl.ds(Rb[d] + b0 * 64, 64)
            tmp = acc_ref[keep1, :] + zrs_recv1[d].astype(F32)
            acc_ref[keep1, :] = tmp
            zrs_send2[d] = acc_ref[
                pl.ds(Rb[d] + b0 * 64 + (1 - b1) * 32, 32), :].astype(BF)
        rdmas = []
        for d in range(2):
            r = pltpu.make_async_remote_copy(
                src_ref=zrs_send2.at[d], dst_ref=zrs_recv2.at[d],
                send_sem=zrs_ssem.at[d, 1], recv_sem=zrs_rsem.at[d, 1],
                device_id=(zh[1],), device_id_type=MESH)
            r.start()
            rdmas.append(r)
        for d in range(2):
            rdmas[d].wait()
        for d in range(2):
            off = pl.ds(Rb[d] + b0 * 64 + b1 * 32, 32)
            tmp = (acc_ref[off, :] + zrs_recv2[d].astype(F32)).astype(BF)
            kv_ref[off, :] = tmp
            zag_send1[d] = tmp

        rdmas = []
        for d in range(2):
            r = pltpu.make_async_remote_copy(
                src_ref=zag_send1.at[d], dst_ref=zag_recv1.at[d],
                send_sem=zag_ssem.at[d, 0], recv_sem=zag_rsem.at[d, 0],
                device_id=(zh[1],), device_id_type=MESH)
            r.start()
            rdmas.append(r)
        for d in range(2):
            rdmas[d].wait()
        for d in range(2):
            kv_ref[pl.ds(Rb[d] + b0 * 64 + (1 - b1) * 32, 32), :] = zag_recv1[d]
            zag_send2[d] = kv_ref[pl.ds(Rb[d] + b0 * 64, 64), :]
        rdmas = []
        for d in range(2):
            r = pltpu.make_async_remote_copy(
                src_ref=zag_send2.at[d], dst_ref=zag_recv2.at[d],
                send_sem=zag_ssem.at[d, 1], recv_sem=zag_rsem.at[d, 1],
                device_id=(zh[0],), device_id_type=MESH)
            r.start()
            rdmas.append(r)
        for d in range(2):
            rdmas[d].wait()
        for d in range(2):
            kv_ref[pl.ds(Rb[d] + (1 - b0) * 64, 64), :] = zag_recv2[d]

        pag_send[...] = kv_ref[pl.ds(T * Q4, Q4), :]
        rdmas = []
        for c in range(2):
            r = pltpu.make_async_remote_copy(
                src_ref=pag_send, dst_ref=pag_recv1.at[c],
                send_sem=pag_ssem.at[c], recv_sem=pag_rsem.at[c],
                device_id=(plane_tgt[c],), device_id_type=MESH)
            r.start()
            rdmas.append(r)
        for c in range(2):
            rdmas[c].wait()
        fwd = []
        for c, half in ((0, pl.ds(0, HQ)), (1, pl.ds(HQ, HQ))):
            r = pltpu.make_async_remote_copy(
                src_ref=pag_recv1.at[c, half], dst_ref=pag_recv2.at[c],
                send_sem=pag_ssem.at[2 + c], recv_sem=pag_rsem.at[2 + c],
                device_id=(plane_tgt[c],), device_id_type=MESH)
            r.start()
            fwd.append(r)
        kv_ref[pl.ds(my_q * Q4, Q4), :] = pag_recv1[0]
        kv_ref[pl.ds(lax.rem(my_q + 2, 4) * Q4, Q4), :] = pag_recv1[1]
        for c in range(2):
            fwd[c].wait()
        qm1 = lax.rem(my_q + 3, 4)
        kv_ref[pl.ds(qm1 * Q4, HQ), :] = pag_recv2[0]
        kv_ref[pl.ds(qm1 * Q4 + HQ, HQ), :] = pag_recv2[1]

        K = kv_ref[0:BS, :]
        V = kv_ref[BS:ROWS, :]
        for b in range(B):
            for h in range(H):
                qh = qm[b * S:(b + 1) * S, h * Dh:(h + 1) * Dh]
                kh = K[b * S:(b + 1) * S, h * Dh:(h + 1) * Dh]
                vh = V[b * S:(b + 1) * S, h * Dh:(h + 1) * Dh]
                sc = (scale * lax.dot_general(qh, kh, nt,
                                              preferred_element_type=F32)
                      + rope_ref[b * S:(b + 1) * S, h * S:(h + 1) * S])
                pr = jnp.exp(sc)
                recip = 1.0 / jnp.sum(pr, axis=-1, keepdims=True)
                ov = jnp.dot(pr.astype(BF), vh, preferred_element_type=F32)
                o_ref[b * S:(b + 1) * S, h * Dh:(h + 1) * Dh] = (
                    ov * recip).astype(BF)
        out = jnp.dot(o_ref[...], wo_ref[...].astype(BF),
                      preferred_element_type=F32)
        out_ref[...] = out.reshape(B, S, D)

        @functools.partial(pl.run_scoped,
                           second_barrier=pltpu.SemaphoreType.REGULAR)
        def _(second_barrier):
            for nbr in (*plane_tgt, *zh):
                pl.semaphore_signal(second_barrier, inc=1, device_id=(nbr,),
                                    device_id_type=MESH)
            pl.semaphore_wait(second_barrier, 4)

    return pl.pallas_call(
        body,
        out_shape=jax.ShapeDtypeStruct((B, S, D), F32),
        in_specs=[pl.BlockSpec(memory_space=pltpu.VMEM)] * 8,
        out_specs=pl.BlockSpec(memory_space=pltpu.VMEM),
        scratch_shapes=[
            pltpu.VMEM((ROWS, D), F32),
            pltpu.VMEM((ROWS, D), BF),
            pltpu.VMEM((BS, D), BF),
            pltpu.VMEM((BS, H * S), F32),
            pltpu.VMEM((2, 3, HQ, D), BF),
            pltpu.VMEM((2, 3, HQ, D), BF),
            pltpu.VMEM((2, 64, D), BF),
            pltpu.VMEM((2, 64, D), BF),
            pltpu.VMEM((2, 32, D), BF),
            pltpu.VMEM((2, 32, D), BF),
            pltpu.VMEM((2, 32, D), BF),
            pltpu.VMEM((2, 32, D), BF),
            pltpu.VMEM((2, 64, D), BF),
            pltpu.VMEM((2, 64, D), BF),
            pltpu.VMEM((Q4, D), BF),
            pltpu.VMEM((2, Q4, D), BF),
            pltpu.VMEM((2, HQ, D), BF),
            pltpu.SemaphoreType.DMA((2, 3)),
            pltpu.SemaphoreType.DMA((2, 3)),
            pltpu.SemaphoreType.DMA((2, 2)),
            pltpu.SemaphoreType.DMA((2, 2)),
            pltpu.SemaphoreType.DMA((2, 2)),
            pltpu.SemaphoreType.DMA((2, 2)),
            pltpu.SemaphoreType.DMA((4,)),
            pltpu.SemaphoreType.DMA((4,)),
        ],
        compiler_params=pltpu.CompilerParams(collective_id=0),
    )(x, Wdkv, Wuk, Wuv, Wq, Wqr, Wkr, Wo)
